# baseline (speedup 1.0000x reference)
import functools

import jax
import jax.numpy as jnp
from jax import lax
from jax.experimental import pallas as pl
from jax.experimental.pallas import tpu as pltpu
from jax.experimental.pallas import tpu_sc as plsc

N_NODES = 10000
D_FEAT = 128
D_EDGE = 16
D_OUT = 16

NC, NS = 2, 16
NW = NC * NS
BLK = 80
TPW = 125
EPW = BLK * TPW
E_TOT = EPW * NW
EROWS = BLK * D_OUT // 128
_EW_BLK = 4000


def _block_diag8(w):
    r, c = w.shape
    big = jnp.concatenate([w] * 8, axis=0)
    big = jnp.concatenate([big] * 8, axis=1)
    rb = lax.broadcasted_iota(jnp.int32, (8 * r, 8 * c), 0) // r
    cb = lax.broadcasted_iota(jnp.int32, (8 * r, 8 * c), 1) // c
    return jnp.where(rb == cb, big, 0.0)


def _tc_body(x8_ref, ew_ref, w1_ref, b_ref, ts_ref, td_ref, o_ref,
             wbd_ref, bt_ref):
    i = pl.program_id(0)

    @pl.when(i == 0)
    def _():
        x8 = x8_ref[...]
        ts_ref[...] = jnp.dot(
            x8, _block_diag8(w1_ref[D_EDGE:D_EDGE + D_FEAT, :]),
            preferred_element_type=jnp.float32)
        td_ref[...] = jnp.dot(
            x8, _block_diag8(w1_ref[D_EDGE + D_FEAT:, :]),
            preferred_element_type=jnp.float32)
        wbd_ref[...] = _block_diag8(w1_ref[:D_EDGE, :])
        bt_ref[...] = jnp.concatenate([b_ref[...]] * 8, axis=1)

    o_ref[...] = (
        jnp.dot(ew_ref[...], wbd_ref[...], preferred_element_type=jnp.float32)
        + bt_ref[...]
    )


def _tc_stage(x8, ew8, w1, b1r):
    rows = E_TOT // 8
    n_ew = rows // _EW_BLK
    return pl.pallas_call(
        _tc_body,
        grid=(n_ew + 1,),
        in_specs=[
            pl.BlockSpec((N_NODES // 8, 8 * D_FEAT), lambda i: (0, 0)),
            pl.BlockSpec((_EW_BLK, 128), lambda i: (jnp.maximum(i - 1, 0), 0)),
            pl.BlockSpec((2 * D_FEAT + D_EDGE, D_OUT), lambda i: (0, 0)),
            pl.BlockSpec((1, D_OUT), lambda i: (0, 0)),
        ],
        out_specs=[
            pl.BlockSpec((N_NODES // 8, 128), lambda i: (0, 0)),
            pl.BlockSpec((N_NODES // 8, 128), lambda i: (0, 0)),
            pl.BlockSpec((_EW_BLK, 128), lambda i: (jnp.maximum(i - 1, 0), 0)),
        ],
        out_shape=[
            jax.ShapeDtypeStruct((N_NODES // 8, 128), jnp.float32),
            jax.ShapeDtypeStruct((N_NODES // 8, 128), jnp.float32),
            jax.ShapeDtypeStruct((rows, 128), jnp.float32),
        ],
        scratch_shapes=[
            pltpu.VMEM((128, 128), jnp.float32),
            pltpu.VMEM((1, 128), jnp.float32),
        ],
    )(x8, ew8, w1, b1r)


def _sc_body(xs_hbm, xd_hbm, ewp_hbm, src_hbm, dst_hbm, out_hbm,
             idx_sv, idx_dv, bufs_e, bufs_s, bufs_d, bufs_o,
             sems_e, sems_s, sems_d, sems_o):
    wid = lax.axis_index("s") * NC + lax.axis_index("c")
    base = wid * EPW
    rbase = wid * (EPW * D_OUT // 128)
    pltpu.sync_copy(src_hbm.at[pl.ds(base, EPW)], idx_sv)
    pltpu.sync_copy(dst_hbm.at[pl.ds(base, EPW)], idx_dv)

    def issue_in(t, b):
        pltpu.async_copy(
            xs_hbm.at[idx_sv.at[pl.ds(t * BLK, BLK)]], bufs_s[b], sems_s[b])
        pltpu.async_copy(
            xd_hbm.at[idx_dv.at[pl.ds(t * BLK, BLK)]], bufs_d[b], sems_d[b])
        pltpu.async_copy(
            ewp_hbm.at[pl.ds(rbase + t * EROWS, EROWS), :],
            bufs_e[b], sems_e[b])

    def drain_in(t, b):
        pltpu.make_async_copy(
            xs_hbm.at[idx_sv.at[pl.ds(t * BLK, BLK)]], bufs_s[b],
            sems_s[b]).wait()
        pltpu.make_async_copy(
            xd_hbm.at[idx_dv.at[pl.ds(t * BLK, BLK)]], bufs_d[b],
            sems_d[b]).wait()
        pltpu.make_async_copy(
            ewp_hbm.at[pl.ds(rbase + t * EROWS, EROWS), :], bufs_e[b],
            sems_e[b]).wait()

    def drain_out(t, b):
        pltpu.make_async_copy(
            bufs_o[b], out_hbm.at[pl.ds(base + t * BLK, BLK), :],
            sems_o[b]).wait()

    def combine(b):
        e_ref, s_ref, d_ref, o_ref = bufs_e[b], bufs_s[b], bufs_d[b], bufs_o[b]

        def row_body(r, c):
            for k in range(8):
                i = r * 8 + k
                o_ref[i, :] = (
                    e_ref[r, pl.ds(k * D_OUT, D_OUT)]
                    + s_ref[i, :] + d_ref[i, :]
                )
            return c

        lax.fori_loop(0, EROWS, row_body, 0, unroll=2)

    def half(t, b):
        issue_in(t + 1, 1 - b)
        drain_in(t, b)
        pl.when(t >= 2)(lambda: drain_out(t - 2, b))
        combine(b)
        pltpu.async_copy(
            bufs_o[b], out_hbm.at[pl.ds(base + t * BLK, BLK), :], sems_o[b])

    issue_in(0, 0)

    def g_body(g, c):
        half(2 * g, 0)
        half(2 * g + 1, 1)
        return c

    lax.fori_loop(0, (TPW - 1) // 2, g_body, 0)
    t_last = TPW - 1
    drain_in(t_last, 0)
    drain_out(t_last - 2, 0)
    combine(0)
    pltpu.async_copy(
        bufs_o[0], out_hbm.at[pl.ds(base + t_last * BLK, BLK), :], sems_o[0])
    drain_out(t_last - 1, 1)
    drain_out(t_last, 0)


@functools.lru_cache(maxsize=None)
def _sc_gather_combine():
    return pl.kernel(
        _sc_body,
        out_type=jax.ShapeDtypeStruct((E_TOT, D_OUT), jnp.float32),
        mesh=plsc.VectorSubcoreMesh(
            core_axis_name="c", subcore_axis_name="s",
            num_cores=NC, num_subcores=NS,
        ),
        scratch_types=[
            pltpu.VMEM((EPW,), jnp.int32),
            pltpu.VMEM((EPW,), jnp.int32),
            [pltpu.VMEM((EROWS, 128), jnp.float32)] * 2,
            [pltpu.VMEM((BLK, D_OUT), jnp.float32)] * 2,
            [pltpu.VMEM((BLK, D_OUT), jnp.float32)] * 2,
            [pltpu.VMEM((BLK, D_OUT), jnp.float32)] * 2,
            [pltpu.SemaphoreType.DMA] * 2,
            [pltpu.SemaphoreType.DMA] * 2,
            [pltpu.SemaphoreType.DMA] * 2,
            [pltpu.SemaphoreType.DMA] * 2,
        ],
        compiler_params=pltpu.CompilerParams(use_tc_tiling_on_sc=False),
    )


def kernel(x, edge_index, edge_weight, W1, b1):
    x8 = x.reshape(N_NODES // 8, 8 * D_FEAT)
    ew8 = edge_weight.reshape(E_TOT // 8, 128)
    b1r = b1.reshape(1, D_OUT)

    ts8, td8, ewp8 = _tc_stage(x8, ew8, W1, b1r)
    xs = ts8.reshape(N_NODES, D_OUT)
    xd = td8.reshape(N_NODES, D_OUT)

    idx = edge_index.astype(jnp.int32)
    src = jnp.minimum(idx[0], N_NODES - 1)
    dst = jnp.minimum(idx[1], N_NODES - 1)
    acc = _sc_gather_combine()(xs, xd, ewp8, src, dst)
    return jnp.maximum(acc, 0.0)


# speedup vs baseline: 1.2495x; 1.2495x over previous
import functools

import jax
import jax.numpy as jnp
from jax import lax
from jax.experimental import pallas as pl
from jax.experimental.pallas import tpu as pltpu
from jax.experimental.pallas import tpu_sc as plsc

N_NODES = 10000
D_FEAT = 128
D_EDGE = 16
D_OUT = 16

NC, NS = 2, 16
NW = NC * NS
BLK = 80
TPW = 125
EPW = BLK * TPW
E_TOT = EPW * NW
EROWS = BLK * D_OUT // 128
_EW_BLK = 4000


def _block_diag8(w):
    r, c = w.shape
    big = jnp.concatenate([w] * 8, axis=0)
    big = jnp.concatenate([big] * 8, axis=1)
    rb = lax.broadcasted_iota(jnp.int32, (8 * r, 8 * c), 0) // r
    cb = lax.broadcasted_iota(jnp.int32, (8 * r, 8 * c), 1) // c
    return jnp.where(rb == cb, big, 0.0)


def _tc_body(x8_ref, ew_ref, w1_ref, b_ref, ts_ref, td_ref, o_ref,
             wbd_ref, bt_ref):
    i = pl.program_id(0)

    @pl.when(i == 0)
    def _():
        x8 = x8_ref[...]
        ts_ref[...] = jnp.dot(
            x8, _block_diag8(w1_ref[D_EDGE:D_EDGE + D_FEAT, :]),
            preferred_element_type=jnp.float32)
        td_ref[...] = jnp.dot(
            x8, _block_diag8(w1_ref[D_EDGE + D_FEAT:, :]),
            preferred_element_type=jnp.float32)
        wbd_ref[...] = _block_diag8(w1_ref[:D_EDGE, :])
        bt_ref[...] = jnp.concatenate([b_ref[...]] * 8, axis=1)

    o_ref[...] = (
        jnp.dot(ew_ref[...], wbd_ref[...], preferred_element_type=jnp.float32)
        + bt_ref[...]
    )


def _tc_stage(x8, ew8, w1, b1r):
    rows = E_TOT // 8
    n_ew = rows // _EW_BLK
    return pl.pallas_call(
        _tc_body,
        grid=(n_ew + 1,),
        in_specs=[
            pl.BlockSpec((N_NODES // 8, 8 * D_FEAT), lambda i: (0, 0)),
            pl.BlockSpec((_EW_BLK, 128), lambda i: (jnp.maximum(i - 1, 0), 0)),
            pl.BlockSpec((2 * D_FEAT + D_EDGE, D_OUT), lambda i: (0, 0)),
            pl.BlockSpec((1, D_OUT), lambda i: (0, 0)),
        ],
        out_specs=[
            pl.BlockSpec((N_NODES // 8, 128), lambda i: (0, 0)),
            pl.BlockSpec((N_NODES // 8, 128), lambda i: (0, 0)),
            pl.BlockSpec((_EW_BLK, 128), lambda i: (jnp.maximum(i - 1, 0), 0)),
        ],
        out_shape=[
            jax.ShapeDtypeStruct((N_NODES // 8, 128), jnp.float32),
            jax.ShapeDtypeStruct((N_NODES // 8, 128), jnp.float32),
            jax.ShapeDtypeStruct((rows, 128), jnp.float32),
        ],
        scratch_shapes=[
            pltpu.VMEM((128, 128), jnp.float32),
            pltpu.VMEM((1, 128), jnp.float32),
        ],
    )(x8, ew8, w1, b1r)


def _sc_body(xs_hbm, xd_hbm, ewp_hbm, ei_hbm, out_hbm,
             idx_sv, idx_dv, bufs_e, bufs_s, bufs_d, bufs_o,
             sems_e, sems_s, sems_d, sems_o):
    wid = lax.axis_index("s") * NC + lax.axis_index("c")
    base = wid * EPW
    rbase = wid * (EPW * D_OUT // 128)
    pltpu.sync_copy(ei_hbm.at[0, pl.ds(base, EPW)], idx_sv)
    pltpu.sync_copy(ei_hbm.at[1, pl.ds(base, EPW)], idx_dv)

    def issue_in(t, b):
        pltpu.async_copy(
            xs_hbm.at[idx_sv.at[pl.ds(t * BLK, BLK)]], bufs_s[b], sems_s[b])
        pltpu.async_copy(
            xd_hbm.at[idx_dv.at[pl.ds(t * BLK, BLK)]], bufs_d[b], sems_d[b])
        pltpu.async_copy(
            ewp_hbm.at[pl.ds(rbase + t * EROWS, EROWS), :],
            bufs_e[b], sems_e[b])

    def drain_in(t, b):
        pltpu.make_async_copy(
            xs_hbm.at[idx_sv.at[pl.ds(t * BLK, BLK)]], bufs_s[b],
            sems_s[b]).wait()
        pltpu.make_async_copy(
            xd_hbm.at[idx_dv.at[pl.ds(t * BLK, BLK)]], bufs_d[b],
            sems_d[b]).wait()
        pltpu.make_async_copy(
            ewp_hbm.at[pl.ds(rbase + t * EROWS, EROWS), :], bufs_e[b],
            sems_e[b]).wait()

    def drain_out(t, b):
        pltpu.make_async_copy(
            bufs_o[b], out_hbm.at[pl.ds(base + t * BLK, BLK), :],
            sems_o[b]).wait()

    def combine(b):
        e_ref, s_ref, d_ref, o_ref = bufs_e[b], bufs_s[b], bufs_d[b], bufs_o[b]

        def row_body(r, c):
            for k in range(8):
                i = r * 8 + k
                o_ref[i, :] = jnp.maximum(
                    e_ref[r, pl.ds(k * D_OUT, D_OUT)]
                    + s_ref[i, :] + d_ref[i, :],
                    0.0,
                )
            return c

        lax.fori_loop(0, EROWS, row_body, 0, unroll=2)

    def half(t, b):
        issue_in(t + 1, 1 - b)
        drain_in(t, b)
        pl.when(t >= 2)(lambda: drain_out(t - 2, b))
        combine(b)
        pltpu.async_copy(
            bufs_o[b], out_hbm.at[pl.ds(base + t * BLK, BLK), :], sems_o[b])

    issue_in(0, 0)

    def g_body(g, c):
        half(2 * g, 0)
        half(2 * g + 1, 1)
        return c

    lax.fori_loop(0, (TPW - 1) // 2, g_body, 0)
    t_last = TPW - 1
    drain_in(t_last, 0)
    drain_out(t_last - 2, 0)
    combine(0)
    pltpu.async_copy(
        bufs_o[0], out_hbm.at[pl.ds(base + t_last * BLK, BLK), :], sems_o[0])
    drain_out(t_last - 1, 1)
    drain_out(t_last, 0)


@functools.lru_cache(maxsize=None)
def _sc_gather_combine():
    return pl.kernel(
        _sc_body,
        out_type=jax.ShapeDtypeStruct((E_TOT, D_OUT), jnp.float32),
        mesh=plsc.VectorSubcoreMesh(
            core_axis_name="c", subcore_axis_name="s",
            num_cores=NC, num_subcores=NS,
        ),
        scratch_types=[
            pltpu.VMEM((EPW,), jnp.int32),
            pltpu.VMEM((EPW,), jnp.int32),
            [pltpu.VMEM((EROWS, 128), jnp.float32)] * 2,
            [pltpu.VMEM((BLK, D_OUT), jnp.float32)] * 2,
            [pltpu.VMEM((BLK, D_OUT), jnp.float32)] * 2,
            [pltpu.VMEM((BLK, D_OUT), jnp.float32)] * 2,
            [pltpu.SemaphoreType.DMA] * 2,
            [pltpu.SemaphoreType.DMA] * 2,
            [pltpu.SemaphoreType.DMA] * 2,
            [pltpu.SemaphoreType.DMA] * 2,
        ],
        compiler_params=pltpu.CompilerParams(use_tc_tiling_on_sc=False),
    )


def kernel(x, edge_index, edge_weight, W1, b1):
    x8 = x.reshape(N_NODES // 8, 8 * D_FEAT)
    ew8 = edge_weight.reshape(E_TOT // 8, 128)
    b1r = b1.reshape(1, D_OUT)

    ts8, td8, ewp8 = _tc_stage(x8, ew8, W1, b1r)
    xs = ts8.reshape(N_NODES, D_OUT)
    xd = td8.reshape(N_NODES, D_OUT)

    return _sc_gather_combine()(xs, xd, ewp8, edge_index.astype(jnp.int32))
